# baseline (speedup 1.0000x reference)
import functools

import jax
import jax.numpy as jnp
from jax import lax
from jax.experimental import pallas as pl
from jax.experimental.pallas import tpu as pltpu
from jax.experimental.pallas import tpu_sc as plsc

_TABLE = (
    -1.0, -0.6961929798126221, -0.5250729918479919, -0.39491742849349976,
    -0.2844412624835968, -0.18477341532707214, -0.09104993939399719, 0.0,
    0.07958030700683594, 0.16093018651008606, 0.24611227214336395,
    0.33791524171829224, 0.44070979952812195, 0.5626168251037598,
    0.7229567170143127, 1.0,
)
_MID = (
    -0.848096489906311, -0.610632985830307, -0.45999521017074585,
    -0.3396793454885483, -0.23460733890533447, -0.13791167736053467,
    -0.045524969696998596, 0.03979015350341797, 0.120255246758461,
    0.203521229326725, 0.2920137569308281, 0.3893125206232071,
    0.5016633123159409, 0.6427867710590363, 0.8614783585071564,
)

BLOCK = 64
_NC, _NS = 2, 16
_NW = _NC * _NS
_CHUNK_ROWS = 16
_SC_ROWS = 768


def _chain(v):
    d = jnp.full(v.shape, jnp.float32(_TABLE[15]), jnp.float32)
    for i in range(14, -1, -1):
        d = jnp.where(v > jnp.float32(_MID[i]), d, jnp.float32(_TABLE[i]))
    return d


def _bf16_bits(x_f32):
    xi = lax.bitcast_convert_type(x_f32, jnp.int32)
    r = xi + 0x7FFF + (lax.shift_right_logical(xi, 16) & 1)
    return lax.shift_right_logical(r, 16)


def _sc_row_block(wv, row, col, iota):
    g = [wv[row, pl.ds(pl.multiple_of(col + 16 * k, 16), 16)]
         for k in range(4)]
    m = jnp.maximum(jnp.maximum(jnp.abs(g[0]), jnp.abs(g[1])),
                    jnp.maximum(jnp.abs(g[2]), jnp.abs(g[3])))
    for st in (8, 4, 2, 1):
        m = jnp.maximum(m, jnp.take(m, iota ^ st))
    rs = jnp.float32(1.0) / m
    return [_bf16_bits(_chain(g[k] * rs) * m) for k in range(4)]


def _sc_dequant_body(w_hbm, out_hbm, wv, ov):
    wid = lax.axis_index("s") * _NC + lax.axis_index("c")
    ncols = w_hbm.shape[1]
    rows_per_w = _SC_ROWS // _NW
    base = wid * rows_per_w

    iota = lax.iota(jnp.int32, 16)
    blocks_per_row = ncols // BLOCK
    pairs_per_chunk = _CHUNK_ROWS // 2

    def do_chunk(c, _):
        r0 = pl.multiple_of(base + c * _CHUNK_ROWS, _CHUNK_ROWS)
        pltpu.sync_copy(w_hbm.at[pl.ds(r0, _CHUNK_ROWS)], wv)

        def do_block(i, _):
            pr = i // blocks_per_row
            col = (i % blocks_per_row) * BLOCK
            lo = _sc_row_block(wv, 2 * pr, col, iota)
            hi = _sc_row_block(wv, 2 * pr + 1, col, iota)
            for k in range(4):
                word = lo[k] | lax.shift_left(hi[k], 16)
                ov[pr, pl.ds(pl.multiple_of(col + 16 * k, 16), 16)] = word
            return 0

        lax.fori_loop(0, pairs_per_chunk * blocks_per_row, do_block, 0)
        pltpu.sync_copy(
            ov,
            out_hbm.at[pl.ds(pl.multiple_of(r0 // 2, pairs_per_chunk),
                             pairs_per_chunk)])
        return 0

    lax.fori_loop(0, rows_per_w // _CHUNK_ROWS, do_chunk, 0)


def _block_maxabs(w):
    a = jnp.abs(w)
    lane = jax.lax.broadcasted_iota(jnp.int32, w.shape, 1)
    n = w.shape[1]
    s = 1
    while s < BLOCK:
        a = jnp.maximum(a, pltpu.roll(a, s, 1))
        s *= 2
    b = jnp.where(lane % BLOCK == BLOCK - 1, a, 0.0)
    s = 1
    while s < BLOCK:
        b = jnp.maximum(b, pltpu.roll(b, n - s, 1))
        s *= 2
    return b


def _tc_dequant_body(w_ref, o_ref):
    w = w_ref[...]
    m = _block_maxabs(w)
    o_ref[...] = (_chain(w / m) * m).astype(jnp.bfloat16)


def _xa_body(x_ref, a_ref, xa_ref, xb_ref):
    xb = x_ref[...].astype(jnp.bfloat16)
    xb_ref[...] = xb
    xa_ref[...] = jax.lax.dot_general(
        xb, a_ref[...], (((1,), (1,)), ((), ())),
        preferred_element_type=jnp.float32).astype(jnp.bfloat16)


def _mm_body(n_sc_tiles, x_ref, wsc_ref, wtc_ref, xa_ref, b_ref, bias_ref,
             o_ref):
    o = pl.program_id(0)
    lora = jax.lax.dot_general(
        xa_ref[...], b_ref[...], (((1,), (1,)), ((), ())),
        preferred_element_type=jnp.float32) + bias_ref[...]

    @pl.when(o < n_sc_tiles)
    def _():
        wd = pltpu.bitcast(wsc_ref[...], jnp.bfloat16)
        o_ref[...] = lora + jax.lax.dot_general(
            x_ref[...], wd, (((1,), (1,)), ((), ())),
            preferred_element_type=jnp.float32)

    @pl.when(o >= n_sc_tiles)
    def _():
        o_ref[...] = lora + jax.lax.dot_general(
            x_ref[...], wtc_ref[...], (((1,), (1,)), ((), ())),
            preferred_element_type=jnp.float32)


def kernel(x, W, lora_A, lora_B, bias):
    S, D = x.shape[1], x.shape[2]
    O = W.shape[0]
    R = lora_A.shape[0]

    sc_dequant = functools.partial(
        pl.kernel,
        mesh=plsc.VectorSubcoreMesh(core_axis_name="c", subcore_axis_name="s"),
        out_type=jax.ShapeDtypeStruct((_SC_ROWS // 2, D), jnp.int32),
        scratch_types=[
            pltpu.VMEM((_CHUNK_ROWS, D), jnp.float32),
            pltpu.VMEM((_CHUNK_ROWS // 2, D), jnp.int32),
        ],
    )(_sc_dequant_body)
    Wp_sc = sc_dequant(W)

    TM = 256
    tc_rows = O - _SC_ROWS
    tc_tile0 = _SC_ROWS // TM
    Wd_tc = pl.pallas_call(
        _tc_dequant_body,
        grid=(tc_rows // TM,),
        in_specs=[pl.BlockSpec((TM, D), lambda i: (i + tc_tile0, 0))],
        out_specs=pl.BlockSpec((TM, D), lambda i: (i, 0)),
        out_shape=jax.ShapeDtypeStruct((tc_rows, D), jnp.bfloat16),
    )(W)

    x2 = x.reshape(S, D)
    A = lora_A.astype(jnp.bfloat16)
    B = lora_B.astype(jnp.bfloat16)

    xa, xb = pl.pallas_call(
        _xa_body,
        out_shape=(jax.ShapeDtypeStruct((S, R), jnp.bfloat16),
                   jax.ShapeDtypeStruct((S, D), jnp.bfloat16)),
    )(x2, A)

    TO = 512
    n_sc_tiles = _SC_ROWS // TO
    n_tc_tiles = tc_rows // TO
    bias2 = bias.reshape(1, O)
    y = pl.pallas_call(
        functools.partial(_mm_body, n_sc_tiles),
        grid=(O // TO,),
        in_specs=[
            pl.BlockSpec((S, D), lambda o: (0, 0)),
            pl.BlockSpec((TO // 2, D),
                         lambda o: (jnp.minimum(o, n_sc_tiles - 1), 0)),
            pl.BlockSpec((TO, D),
                         lambda o: (jnp.clip(o - n_sc_tiles, 0,
                                             n_tc_tiles - 1), 0)),
            pl.BlockSpec((S, R), lambda o: (0, 0)),
            pl.BlockSpec((TO, R), lambda o: (o, 0)),
            pl.BlockSpec((1, TO), lambda o: (0, o)),
        ],
        out_specs=pl.BlockSpec((S, TO), lambda o: (0, o)),
        out_shape=jax.ShapeDtypeStruct((S, O), jnp.float32),
        compiler_params=pltpu.CompilerParams(
            dimension_semantics=("arbitrary",)),
    )(xb, Wp_sc, Wd_tc, xa, B, bias2)

    return y.reshape(1, S, O)


# speedup vs baseline: 1.1049x; 1.1049x over previous
import functools

import jax
import jax.numpy as jnp
from jax import lax
from jax.experimental import pallas as pl
from jax.experimental.pallas import tpu as pltpu
from jax.experimental.pallas import tpu_sc as plsc

_TABLE = (
    -1.0, -0.6961929798126221, -0.5250729918479919, -0.39491742849349976,
    -0.2844412624835968, -0.18477341532707214, -0.09104993939399719, 0.0,
    0.07958030700683594, 0.16093018651008606, 0.24611227214336395,
    0.33791524171829224, 0.44070979952812195, 0.5626168251037598,
    0.7229567170143127, 1.0,
)
_MID = (
    -0.848096489906311, -0.610632985830307, -0.45999521017074585,
    -0.3396793454885483, -0.23460733890533447, -0.13791167736053467,
    -0.045524969696998596, 0.03979015350341797, 0.120255246758461,
    0.203521229326725, 0.2920137569308281, 0.3893125206232071,
    0.5016633123159409, 0.6427867710590363, 0.8614783585071564,
)

BLOCK = 64
_NC, _NS = 2, 16
_NW = _NC * _NS
_CHUNK_ROWS = 16
_SC_ROWS = 1280


def _chain(v):
    d = jnp.full(v.shape, jnp.float32(_TABLE[15]), jnp.float32)
    for i in range(14, -1, -1):
        d = jnp.where(v > jnp.float32(_MID[i]), d, jnp.float32(_TABLE[i]))
    return d


def _bf16_bits(x_f32):
    xi = lax.bitcast_convert_type(x_f32, jnp.int32)
    r = xi + 0x7FFF + (lax.shift_right_logical(xi, 16) & 1)
    return lax.shift_right_logical(r, 16)


def _sc_row_block(wv, row, col, iota):
    g = [wv[row, pl.ds(pl.multiple_of(col + 16 * k, 16), 16)]
         for k in range(4)]
    m = jnp.maximum(jnp.maximum(jnp.abs(g[0]), jnp.abs(g[1])),
                    jnp.maximum(jnp.abs(g[2]), jnp.abs(g[3])))
    for st in (8, 4, 2, 1):
        m = jnp.maximum(m, jnp.take(m, iota ^ st))
    rs = jnp.float32(1.0) / m
    return [_bf16_bits(_chain(g[k] * rs) * m) for k in range(4)]


def _sc_dequant_body(w_hbm, out_hbm, wv, ov):
    wid = lax.axis_index("s") * _NC + lax.axis_index("c")
    ncols = w_hbm.shape[1]
    rows_per_w = _SC_ROWS // _NW
    base = wid * rows_per_w

    iota = lax.iota(jnp.int32, 16)
    blocks_per_row = ncols // BLOCK
    pairs_per_chunk = _CHUNK_ROWS // 2

    def do_chunk(c, _):
        r0 = pl.multiple_of(base + c * _CHUNK_ROWS, _CHUNK_ROWS)
        pltpu.sync_copy(w_hbm.at[pl.ds(r0, _CHUNK_ROWS)], wv)

        def do_block(i, _):
            pr = i // blocks_per_row
            col = (i % blocks_per_row) * BLOCK
            lo = _sc_row_block(wv, 2 * pr, col, iota)
            hi = _sc_row_block(wv, 2 * pr + 1, col, iota)
            for k in range(4):
                word = lo[k] | lax.shift_left(hi[k], 16)
                ov[pr, pl.ds(pl.multiple_of(col + 16 * k, 16), 16)] = word
            return 0

        lax.fori_loop(0, pairs_per_chunk * blocks_per_row, do_block, 0)
        pltpu.sync_copy(
            ov,
            out_hbm.at[pl.ds(pl.multiple_of(r0 // 2, pairs_per_chunk),
                             pairs_per_chunk)])
        return 0

    lax.fori_loop(0, rows_per_w // _CHUNK_ROWS, do_chunk, 0)


def _block_maxabs(w):
    a = jnp.abs(w)
    lane = jax.lax.broadcasted_iota(jnp.int32, w.shape, 1)
    n = w.shape[1]
    s = 1
    while s < BLOCK:
        a = jnp.maximum(a, pltpu.roll(a, s, 1))
        s *= 2
    b = jnp.where(lane % BLOCK == BLOCK - 1, a, 0.0)
    s = 1
    while s < BLOCK:
        b = jnp.maximum(b, pltpu.roll(b, n - s, 1))
        s *= 2
    return b


def _tc_dequant_body(w_ref, o_ref):
    w = w_ref[...]
    m = _block_maxabs(w)
    o_ref[...] = (_chain(w / m) * m).astype(jnp.bfloat16)


def _xa_body(x_ref, a_ref, xa_ref, xb_ref):
    xb = x_ref[...].astype(jnp.bfloat16)
    xb_ref[...] = xb
    xa_ref[...] = jax.lax.dot_general(
        xb, a_ref[...], (((1,), (1,)), ((), ())),
        preferred_element_type=jnp.float32).astype(jnp.bfloat16)


def _mm_body(n_sc_tiles, x_ref, wsc_ref, wtc_ref, xa_ref, b_ref, bias_ref,
             o_ref):
    o = pl.program_id(0)
    lora = jax.lax.dot_general(
        xa_ref[...], b_ref[...], (((1,), (1,)), ((), ())),
        preferred_element_type=jnp.float32) + bias_ref[...]

    @pl.when(o < n_sc_tiles)
    def _():
        wd = pltpu.bitcast(wsc_ref[...], jnp.bfloat16)
        o_ref[...] = lora + jax.lax.dot_general(
            x_ref[...], wd, (((1,), (1,)), ((), ())),
            preferred_element_type=jnp.float32)

    @pl.when(o >= n_sc_tiles)
    def _():
        o_ref[...] = lora + jax.lax.dot_general(
            x_ref[...], wtc_ref[...], (((1,), (1,)), ((), ())),
            preferred_element_type=jnp.float32)


def kernel(x, W, lora_A, lora_B, bias):
    S, D = x.shape[1], x.shape[2]
    O = W.shape[0]
    R = lora_A.shape[0]

    sc_dequant = functools.partial(
        pl.kernel,
        mesh=plsc.VectorSubcoreMesh(core_axis_name="c", subcore_axis_name="s"),
        out_type=jax.ShapeDtypeStruct((_SC_ROWS // 2, D), jnp.int32),
        scratch_types=[
            pltpu.VMEM((_CHUNK_ROWS, D), jnp.float32),
            pltpu.VMEM((_CHUNK_ROWS // 2, D), jnp.int32),
        ],
    )(_sc_dequant_body)
    Wp_sc = sc_dequant(W)

    TM = 256
    tc_rows = O - _SC_ROWS
    tc_tile0 = _SC_ROWS // TM
    Wd_tc = pl.pallas_call(
        _tc_dequant_body,
        grid=(tc_rows // TM,),
        in_specs=[pl.BlockSpec((TM, D), lambda i: (i + tc_tile0, 0))],
        out_specs=pl.BlockSpec((TM, D), lambda i: (i, 0)),
        out_shape=jax.ShapeDtypeStruct((tc_rows, D), jnp.bfloat16),
    )(W)

    x2 = x.reshape(S, D)
    A = lora_A.astype(jnp.bfloat16)
    B = lora_B.astype(jnp.bfloat16)

    xa, xb = pl.pallas_call(
        _xa_body,
        out_shape=(jax.ShapeDtypeStruct((S, R), jnp.bfloat16),
                   jax.ShapeDtypeStruct((S, D), jnp.bfloat16)),
    )(x2, A)

    TO = 512
    n_sc_tiles = _SC_ROWS // TO
    n_tc_tiles = tc_rows // TO
    bias2 = bias.reshape(1, O)
    y = pl.pallas_call(
        functools.partial(_mm_body, n_sc_tiles),
        grid=(O // TO,),
        in_specs=[
            pl.BlockSpec((S, D), lambda o: (0, 0)),
            pl.BlockSpec((TO // 2, D),
                         lambda o: (jnp.minimum(o, n_sc_tiles - 1), 0)),
            pl.BlockSpec((TO, D),
                         lambda o: (jnp.clip(o - n_sc_tiles, 0,
                                             n_tc_tiles - 1), 0)),
            pl.BlockSpec((S, R), lambda o: (0, 0)),
            pl.BlockSpec((TO, R), lambda o: (o, 0)),
            pl.BlockSpec((1, TO), lambda o: (0, o)),
        ],
        out_specs=pl.BlockSpec((S, TO), lambda o: (0, o)),
        out_shape=jax.ShapeDtypeStruct((S, O), jnp.float32),
        compiler_params=pltpu.CompilerParams(
            dimension_semantics=("arbitrary",)),
    )(xb, Wp_sc, Wd_tc, xa, B, bias2)

    return y.reshape(1, S, O)
